# baseline (speedup 1.0000x reference)
import functools

import jax
import jax.numpy as jnp
from jax import lax
from jax.experimental import pallas as pl
from jax.experimental.pallas import tpu as pltpu
from jax.experimental.pallas import tpu_sc as plsc

N = 10000
U = 2500
D = 256
E_U = 160000
E_R = 10000
E_H = 20000
PAD = 1152

NC = 2
NS = 16
NW = NC * NS
B = 64
SB = 16
LCH = D // 16

RN = 320
RN_LAST = N - (NW - 1) * RN
RU = 160
U6_ROWS = NS * RU
GARB = RN
SENTINEL = 1 << 30


def _linear_body(x_ref, w_ref, b_ref, o_ref):
    o_ref[...] = lax.dot_general(
        x_ref[...], w_ref[...], (((1,), (1,)), ((), ())),
        preferred_element_type=jnp.float32) + b_ref[...]


def _linear(x, W, b):
    m_blk = 1000
    return pl.pallas_call(
        _linear_body,
        grid=(N // m_blk,),
        in_specs=[
            pl.BlockSpec((m_blk, D), lambda i: (i, 0)),
            pl.BlockSpec((D, D), lambda i: (0, 0)),
            pl.BlockSpec((1, D), lambda i: (0, 0)),
        ],
        out_specs=pl.BlockSpec((m_blk, D), lambda i: (i, 0)),
        out_shape=jax.ShapeDtypeStruct((N, D), jnp.float32),
    )(x, W, b.reshape(1, D))


def _read_split(pbuf, nch, i):
    tot = jnp.int32(0)
    for ch in range(nch):
        v = pbuf[pl.ds(ch * 16, 16)]
        for l in range(16):
            tot = jnp.where(jnp.int32(ch * 16 + l) == i, v[l], tot)
    return tot


def _sc_body(h_hbm, us_hbm, ud_hbm, rs_hbm, rd_hbm, hs_hbm, hd_hbm,
             pu_hbm, pr_hbm, ph_hbm,
             out_hbm, out6_hbm, idx_s, idx_d, rows0, rows1, acc,
             pu_v, pr_v, ph_v, sem0, sem1):
    c = lax.axis_index("c")
    s = lax.axis_index("s")
    w = c * NS + s

    pltpu.sync_copy(pu_hbm, pu_v)
    pltpu.sync_copy(pr_hbm, pr_v)
    pltpu.sync_copy(ph_hbm, ph_v)

    def zero_acc(nrows):
        def zb(t, carry):
            i = t // LCH
            k = t % LCH
            acc[i, pl.ds(k * 16, 16)] = jnp.zeros((16,), jnp.float32)
            return carry
        lax.fori_loop(0, nrows * LCH, zb, 0)

    def run(src_hbm, dst_hbm, table, lo, hi, row_base, src_off=None):
        base0 = (lo // 8) * 8
        nb = (hi - base0 + (B - 1)) // B
        nsb = (nb + SB - 1) // SB
        bufs = [rows0, rows1]
        sems = [sem0, sem1]

        def process(j, sbase, par):
            buf = bufs[par]
            pltpu.make_async_copy(table.at[pl.ds(0, B)], buf,
                                  sems[par]).wait()

            def gbody(g, carry2):
                off = j * B + g * 16
                d = idx_d[pl.ds(off, 16)]
                pos = lax.iota(jnp.int32, 16) + (sbase + off)
                valid = (pos >= lo) & (pos < hi)
                loc = jnp.where(valid, d - row_base, GARB)
                locs = [loc[l] for l in range(16)]
                run = [buf[g * 16, pl.ds(i2 * 16, 16)] for i2 in range(LCH)]
                for l in range(16):
                    last = (l == 15)
                    loc_l = locs[l]
                    if last:
                        for i2 in range(LCH):
                            plsc.addupdate(acc.at[loc_l, pl.ds(i2 * 16, 16)],
                                           run[i2])
                    else:
                        fl = locs[l] != locs[l + 1]

                        @pl.when(fl)
                        def _(run=run, loc_l=loc_l):
                            for i2 in range(LCH):
                                plsc.addupdate(
                                    acc.at[loc_l, pl.ds(i2 * 16, 16)],
                                    run[i2])
                    if not last:
                        row = g * 16 + l + 1
                        nxt = [buf[row, pl.ds(i2 * 16, 16)]
                               for i2 in range(LCH)]
                        run = [jnp.where(fl, nxt[i2], run[i2] + nxt[i2])
                               for i2 in range(LCH)]
                return carry2
            lax.fori_loop(0, B // 16, gbody, 0)

        def sbody(m, carry):
            sbase = base0 + m * SB * B
            pltpu.sync_copy(src_hbm.at[pl.ds(sbase, SB * B)], idx_s)
            pltpu.sync_copy(dst_hbm.at[pl.ds(sbase, SB * B)], idx_d)
            if src_off is not None:
                for k in range(SB * B // 16):
                    sv = idx_s[pl.ds(k * 16, 16)]
                    idx_s[pl.ds(k * 16, 16)] = sv + src_off
            pltpu.async_copy(table.at[idx_s.at[pl.ds(0, B)]], rows0, sem0)

            def bbody(j, carry2):
                for par in (0, 1):
                    @pl.when((j % 2) == par)
                    def _(par=par):
                        @pl.when(j + 1 < SB)
                        def _():
                            pltpu.async_copy(
                                table.at[idx_s.at[pl.ds((j + 1) * B, B)]],
                                bufs[1 - par], sems[1 - par])
                        process(j, sbase, par)
                return carry2
            lax.fori_loop(0, SB, bbody, 0)
            return carry
        lax.fori_loop(0, nsb, sbody, 0)

    zero_acc(RN)
    r_lo = _read_split(pr_v, 2, s)
    r_hi = _read_split(pr_v, 2, s + 1)
    run(rs_hbm, rd_hbm, h_hbm, r_lo, r_hi, row_base=s * RU)
    pltpu.sync_copy(acc.at[pl.ds(0, RU)],
                    out6_hbm.at[pl.ds(c * U6_ROWS + s * RU, RU)])
    zero_acc(RU)
    plsc.subcore_barrier()

    u_lo = _read_split(pu_v, 3, w)
    u_hi = _read_split(pu_v, 3, w + 1)
    run(us_hbm, ud_hbm, h_hbm, u_lo, u_hi, row_base=w * RN)

    h_lo = _read_split(ph_v, 3, w)
    h_hi = _read_split(ph_v, 3, w + 1)
    run(hs_hbm, hd_hbm, out6_hbm, h_lo, h_hi, row_base=w * RN,
        src_off=c * U6_ROWS)

    @pl.when(w < NW - 1)
    def _():
        pltpu.sync_copy(acc.at[pl.ds(0, RN)],
                        out_hbm.at[pl.ds(w * RN, RN)])

    @pl.when(w == NW - 1)
    def _():
        pltpu.sync_copy(acc.at[pl.ds(0, RN_LAST)],
                        out_hbm.at[pl.ds((NW - 1) * RN, RN_LAST)])


@functools.cache
def _sc_call():
    mesh = plsc.VectorSubcoreMesh(core_axis_name="c", subcore_axis_name="s")
    return pl.kernel(
        _sc_body,
        out_type=[jax.ShapeDtypeStruct((N, D), jnp.float32),
                  jax.ShapeDtypeStruct((NC * U6_ROWS, D), jnp.float32)],
        mesh=mesh,
        scratch_types=[
            pltpu.VMEM((SB * B,), jnp.int32),
            pltpu.VMEM((SB * B,), jnp.int32),
            pltpu.VMEM((B, D), jnp.float32),
            pltpu.VMEM((B, D), jnp.float32),
            pltpu.VMEM((RN + 8, D), jnp.float32),
            pltpu.VMEM((48,), jnp.int32),
            pltpu.VMEM((32,), jnp.int32),
            pltpu.VMEM((48,), jnp.int32),
            pltpu.SemaphoreType.DMA,
            pltpu.SemaphoreType.DMA,
        ],
    )


def _pad_idx(src, dst, n):
    i32 = jnp.int32
    pad = n - src.shape[0]
    return (jnp.concatenate([src.astype(i32), jnp.zeros((pad,), i32)]),
            jnp.concatenate([dst.astype(i32), jnp.full((pad,), SENTINEL,
                                                       i32)]))


def kernel(x, W, b, u_src, u_dst, r_src, r_dst, h_src, h_dst):
    i32 = jnp.int32
    h = _linear(x, W, b)
    us, ud = _pad_idx(u_src, u_dst, E_U + PAD)
    rs, rd = _pad_idx(r_src, r_dst, E_R + PAD)
    hs, hd = _pad_idx(h_src, h_dst, E_H + PAD)
    udc = u_dst.astype(i32)
    rdc = r_dst.astype(i32)
    hdc = h_dst.astype(i32)
    bounds_n = jnp.arange(NW + 1, dtype=i32) * RN
    bounds_u = jnp.arange(NS + 1, dtype=i32) * RU
    pu = jnp.zeros((48,), i32).at[: NW + 1].set(
        jnp.searchsorted(udc, bounds_n).astype(i32))
    pr = jnp.zeros((32,), i32).at[: NS + 1].set(
        jnp.searchsorted(rdc, bounds_u).astype(i32))
    ph = jnp.zeros((48,), i32).at[: NW + 1].set(
        jnp.searchsorted(hdc, bounds_n).astype(i32))
    out, _ = _sc_call()(h, us, ud, rs, rd, hs, hd, pu, pr, ph)
    return out


# speedup vs baseline: 1.1455x; 1.1455x over previous
import functools

import jax
import jax.numpy as jnp
from jax import lax
from jax.experimental import pallas as pl
from jax.experimental.pallas import tpu as pltpu
from jax.experimental.pallas import tpu_sc as plsc

N = 10000
U = 2500
D = 256
E_U = 160000
E_R = 10000
E_H = 20000
PAD = 640

NC = 2
NS = 16
NW = NC * NS
B = 64
SB = 8
LCH = D // 16

RN = 320
RN_LAST = N - (NW - 1) * RN
RU = 160
U6_ROWS = NS * RU
GARB = RN
SENTINEL = 1 << 30


def _linear_body(x_ref, w_ref, b_ref, o_ref):
    o_ref[...] = lax.dot_general(
        x_ref[...], w_ref[...], (((1,), (1,)), ((), ())),
        preferred_element_type=jnp.float32) + b_ref[...]


def _linear(x, W, b):
    m_blk = 1000
    return pl.pallas_call(
        _linear_body,
        grid=(N // m_blk,),
        in_specs=[
            pl.BlockSpec((m_blk, D), lambda i: (i, 0)),
            pl.BlockSpec((D, D), lambda i: (0, 0)),
            pl.BlockSpec((1, D), lambda i: (0, 0)),
        ],
        out_specs=pl.BlockSpec((m_blk, D), lambda i: (i, 0)),
        out_shape=jax.ShapeDtypeStruct((N, D), jnp.float32),
    )(x, W, b.reshape(1, D))


def _read_split(pbuf, nch, i):
    tot = jnp.int32(0)
    for ch in range(nch):
        v = pbuf[pl.ds(ch * 16, 16)]
        for l in range(16):
            tot = jnp.where(jnp.int32(ch * 16 + l) == i, v[l], tot)
    return tot


def _sc_body(h_hbm, us_hbm, ud_hbm, rs_hbm, rd_hbm, hs_hbm, hd_hbm,
             pu_hbm, pr_hbm, ph_hbm,
             out_hbm, out6_hbm, idx_s, idx_d, rows0, rows1, acc,
             pu_v, pr_v, ph_v, sem0, sem1):
    c = lax.axis_index("c")
    s = lax.axis_index("s")
    w = c * NS + s

    pltpu.sync_copy(pu_hbm, pu_v)
    pltpu.sync_copy(pr_hbm, pr_v)
    pltpu.sync_copy(ph_hbm, ph_v)

    def zero_acc(nrows):
        def zb(t, carry):
            i = t // LCH
            k = t % LCH
            acc[i, pl.ds(k * 16, 16)] = jnp.zeros((16,), jnp.float32)
            return carry
        lax.fori_loop(0, nrows * LCH, zb, 0)

    def run(src_hbm, dst_hbm, table, lo, hi, row_base, src_off=None):
        base0 = (lo // 8) * 8
        nb = (hi - base0 + (B - 1)) // B
        nsb = (nb + SB - 1) // SB
        bufs = [rows0, rows1]
        sems = [sem0, sem1]

        def process(j, sbase, par):
            buf = bufs[par]
            pltpu.make_async_copy(table.at[pl.ds(0, B)], buf,
                                  sems[par]).wait()

            def gbody(g, carry2):
                off = j * B + g * 16
                d = idx_d[pl.ds(off, 16)]
                pos = lax.iota(jnp.int32, 16) + (sbase + off)
                valid = (pos >= lo) & (pos < hi)
                loc = jnp.where(valid, d - row_base, GARB)
                locs = [loc[l] for l in range(16)]
                run = [buf[g * 16, pl.ds(i2 * 16, 16)] for i2 in range(LCH)]
                for l in range(16):
                    last = (l == 15)
                    loc_l = locs[l]
                    if last:
                        for i2 in range(LCH):
                            plsc.addupdate(acc.at[loc_l, pl.ds(i2 * 16, 16)],
                                           run[i2])
                    else:
                        fl = locs[l] != locs[l + 1]

                        @pl.when(fl)
                        def _(run=run, loc_l=loc_l):
                            for i2 in range(LCH):
                                plsc.addupdate(
                                    acc.at[loc_l, pl.ds(i2 * 16, 16)],
                                    run[i2])
                    if not last:
                        row = g * 16 + l + 1
                        nxt = [buf[row, pl.ds(i2 * 16, 16)]
                               for i2 in range(LCH)]
                        run = [jnp.where(fl, nxt[i2], run[i2] + nxt[i2])
                               for i2 in range(LCH)]
                return carry2
            lax.fori_loop(0, B // 16, gbody, 0)

        def sbody(m, carry):
            sbase = base0 + m * SB * B
            pltpu.sync_copy(src_hbm.at[pl.ds(sbase, SB * B)], idx_s)
            pltpu.sync_copy(dst_hbm.at[pl.ds(sbase, SB * B)], idx_d)
            if src_off is not None:
                for k in range(SB * B // 16):
                    sv = idx_s[pl.ds(k * 16, 16)]
                    idx_s[pl.ds(k * 16, 16)] = sv + src_off
            pltpu.async_copy(table.at[idx_s.at[pl.ds(0, B)]], rows0, sem0)

            def bbody(j, carry2):
                for par in (0, 1):
                    @pl.when((j % 2) == par)
                    def _(par=par):
                        @pl.when(j + 1 < SB)
                        def _():
                            pltpu.async_copy(
                                table.at[idx_s.at[pl.ds((j + 1) * B, B)]],
                                bufs[1 - par], sems[1 - par])
                        process(j, sbase, par)
                return carry2
            lax.fori_loop(0, SB, bbody, 0)
            return carry
        lax.fori_loop(0, nsb, sbody, 0)

    zero_acc(RN)
    r_lo = _read_split(pr_v, 2, s)
    r_hi = _read_split(pr_v, 2, s + 1)
    run(rs_hbm, rd_hbm, h_hbm, r_lo, r_hi, row_base=s * RU)
    pltpu.sync_copy(acc.at[pl.ds(0, RU)],
                    out6_hbm.at[pl.ds(c * U6_ROWS + s * RU, RU)])
    zero_acc(RU)
    plsc.subcore_barrier()

    u_lo = _read_split(pu_v, 3, w)
    u_hi = _read_split(pu_v, 3, w + 1)
    run(us_hbm, ud_hbm, h_hbm, u_lo, u_hi, row_base=w * RN)

    h_lo = _read_split(ph_v, 3, w)
    h_hi = _read_split(ph_v, 3, w + 1)
    run(hs_hbm, hd_hbm, out6_hbm, h_lo, h_hi, row_base=w * RN,
        src_off=c * U6_ROWS)

    @pl.when(w < NW - 1)
    def _():
        pltpu.sync_copy(acc.at[pl.ds(0, RN)],
                        out_hbm.at[pl.ds(w * RN, RN)])

    @pl.when(w == NW - 1)
    def _():
        pltpu.sync_copy(acc.at[pl.ds(0, RN_LAST)],
                        out_hbm.at[pl.ds((NW - 1) * RN, RN_LAST)])


@functools.cache
def _sc_call():
    mesh = plsc.VectorSubcoreMesh(core_axis_name="c", subcore_axis_name="s")
    return pl.kernel(
        _sc_body,
        out_type=[jax.ShapeDtypeStruct((N, D), jnp.float32),
                  jax.ShapeDtypeStruct((NC * U6_ROWS, D), jnp.float32)],
        mesh=mesh,
        scratch_types=[
            pltpu.VMEM((SB * B,), jnp.int32),
            pltpu.VMEM((SB * B,), jnp.int32),
            pltpu.VMEM((B, D), jnp.float32),
            pltpu.VMEM((B, D), jnp.float32),
            pltpu.VMEM((RN + 8, D), jnp.float32),
            pltpu.VMEM((48,), jnp.int32),
            pltpu.VMEM((32,), jnp.int32),
            pltpu.VMEM((48,), jnp.int32),
            pltpu.SemaphoreType.DMA,
            pltpu.SemaphoreType.DMA,
        ],
    )


def _pad_idx(src, dst, n):
    i32 = jnp.int32
    pad = n - src.shape[0]
    return (jnp.concatenate([src.astype(i32), jnp.zeros((pad,), i32)]),
            jnp.concatenate([dst.astype(i32), jnp.full((pad,), SENTINEL,
                                                       i32)]))


def kernel(x, W, b, u_src, u_dst, r_src, r_dst, h_src, h_dst):
    i32 = jnp.int32
    h = _linear(x, W, b)
    us, ud = _pad_idx(u_src, u_dst, E_U + PAD)
    rs, rd = _pad_idx(r_src, r_dst, E_R + PAD)
    hs, hd = _pad_idx(h_src, h_dst, E_H + PAD)
    udc = u_dst.astype(i32)
    rdc = r_dst.astype(i32)
    hdc = h_dst.astype(i32)
    bounds_n = jnp.arange(NW + 1, dtype=i32) * RN
    bounds_u = jnp.arange(NS + 1, dtype=i32) * RU
    pu = jnp.zeros((48,), i32).at[: NW + 1].set(
        jnp.searchsorted(udc, bounds_n).astype(i32))
    pr = jnp.zeros((32,), i32).at[: NS + 1].set(
        jnp.searchsorted(rdc, bounds_u).astype(i32))
    ph = jnp.zeros((48,), i32).at[: NW + 1].set(
        jnp.searchsorted(hdc, bounds_n).astype(i32))
    out, _ = _sc_call()(h, us, ud, rs, rd, hs, hd, pu, pr, ph)
    return out
